# baseline (speedup 1.0000x reference)
import functools

import jax
import jax.numpy as jnp
from jax import lax
from jax.experimental import pallas as pl
from jax.experimental.pallas import tpu as pltpu
from jax.experimental.pallas import tpu_sc as plsc

T = 2048
D = 1024
I = 512
E = 16
K = 8
N_GROUP = 8
TOPK_GROUP = 4
RSF = 2.5

NC = 2
NS = 16
NW = NC * NS
TPW = T // NW
NSLOT = 4
TB = 256
NTB = T // TB
BR = 256
NBLK = (NSLOT * T) // BR + N_GROUP
PPAD = NBLK * BR
CH = 32


def _route_a_body(x_ref, gw_ref, gb_ref, gmask_ref, w_ref, counts_ref):
    xb = x_ref[...].astype(jnp.bfloat16)
    gwb = gw_ref[...].astype(jnp.bfloat16)
    logits = jax.lax.dot_general(
        xb, gwb, (((1,), (1,)), ((), ())),
        preferred_element_type=jnp.float32)
    scores = jax.nn.sigmoid(logits)
    sfc = scores + gb_ref[...]

    ecol = jax.lax.broadcasted_iota(jnp.int32, (TB, E), 1)
    gs = jnp.concatenate(
        [jnp.sum(jnp.where(ecol // 2 == g, sfc, 0.0), axis=1, keepdims=True)
         for g in range(N_GROUP)], axis=1)

    col = jax.lax.broadcasted_iota(jnp.int32, (TB, N_GROUP), 1)
    rank = jnp.zeros((TB, N_GROUP), jnp.int32)
    for gp in range(N_GROUP):
        other = gs[:, gp:gp + 1]
        gt = (other > gs).astype(jnp.int32)
        tie = jnp.logical_and(other == gs, gp < col).astype(jnp.int32)
        rank = rank + gt + tie
    gmask = (rank < TOPK_GROUP).astype(jnp.float32)

    e_iota = jax.lax.broadcasted_iota(jnp.int32, (N_GROUP, E), 0)
    g_iota = jax.lax.broadcasted_iota(jnp.int32, (N_GROUP, E), 1)
    pmt = (g_iota // 2 == e_iota).astype(jnp.float32)
    emask = jax.lax.dot_general(
        gmask, pmt, (((1,), (0,)), ((), ())),
        preferred_element_type=jnp.float32)

    w = scores * emask
    denom = jnp.sum(w, axis=1, keepdims=True) + 1e-20
    gmask_ref[...] = gmask
    w_ref[...] = w * (RSF / denom)
    counts_ref[0] = jnp.sum(gmask, axis=0, keepdims=True)


def _route_b_body(counts_ref, gmask_ref, w_ref, dpos_ref, wpa_ref, wpb_ref,
                  gid_ref):
    i = pl.program_id(0)
    gmask = gmask_ref[...]
    w = w_ref[...]
    cnts = counts_ref[:, 0, :]

    tot = jnp.sum(cnts, axis=0, keepdims=True)
    bc = jax.lax.shift_right_logical(
        tot.astype(jnp.int32) + (BR - 1), 8)
    bcf = bc.astype(jnp.float32)
    ga = jax.lax.broadcasted_iota(jnp.int32, (N_GROUP, N_GROUP), 0)
    gb2 = jax.lax.broadcasted_iota(jnp.int32, (N_GROUP, N_GROUP), 1)
    hi = jax.lax.Precision.HIGHEST
    base_pad = jax.lax.dot_general(
        bcf, (ga < gb2).astype(jnp.float32), (((1,), (0,)), ((), ())),
        preferred_element_type=jnp.float32, precision=hi) * float(BR)

    brow = jax.lax.broadcasted_iota(jnp.int32, (NTB, 1), 0)
    pre = jnp.sum(jnp.where(brow < i, cnts, 0.0), axis=0,
                  keepdims=True)

    ri = jax.lax.broadcasted_iota(jnp.int32, (TB, 1), 0)
    ci = jax.lax.broadcasted_iota(jnp.int32, (1, TB), 1)
    tril = (ci < ri).astype(jnp.bfloat16)
    rank_in = jax.lax.dot_general(
        tril, gmask.astype(jnp.bfloat16), (((1,), (0,)), ((), ())),
        preferred_element_type=jnp.float32)
    pos = base_pad + pre + rank_in

    col = jax.lax.broadcasted_iota(jnp.int32, (TB, N_GROUP), 1)
    slotidx = jnp.zeros((TB, N_GROUP), jnp.float32)
    for gp in range(1, N_GROUP):
        slotidx = slotidx + jnp.where(col >= gp, gmask[:, gp - 1:gp], 0.0)

    dcols, acols, bcols = [], [], []
    for s in range(NSLOT):
        m_s = jnp.where(jnp.logical_and(slotidx == s, gmask > 0), 1.0, 0.0)
        dcols.append(jnp.sum(m_s * pos, axis=1, keepdims=True))
        wa_s = jnp.zeros((TB, 1), jnp.float32)
        wb_s = jnp.zeros((TB, 1), jnp.float32)
        for g in range(N_GROUP):
            mg = m_s[:, g:g + 1]
            wa_s = wa_s + mg * w[:, 2 * g:2 * g + 1]
            wb_s = wb_s + mg * w[:, 2 * g + 1:2 * g + 2]
        acols.append(wa_s)
        bcols.append(wb_s)
    dpos_ref[...] = jnp.transpose(
        jnp.concatenate(dcols, axis=1)).astype(jnp.int32)
    wpa_ref[...] = jnp.transpose(jnp.concatenate(acols, axis=1))
    wpb_ref[...] = jnp.transpose(jnp.concatenate(bcols, axis=1))

    @pl.when(i == 0)
    def _gid():
        cbci = jax.lax.dot_general(
            bcf, (ga <= gb2).astype(jnp.float32), (((1,), (0,)), ((), ())),
            preferred_element_type=jnp.float32, precision=hi)
        blk = jax.lax.broadcasted_iota(jnp.int32, (1, 64), 1)
        gid = jnp.zeros((1, 64), jnp.int32)
        g_sel = jax.lax.broadcasted_iota(jnp.int32, (1, N_GROUP), 1)
        for g in range(N_GROUP):
            cg = jnp.sum(jnp.where(g_sel == g, cbci, 0.0)).astype(jnp.int32)
            gid = gid + (blk >= cg).astype(jnp.int32)
        gid_ref[...] = jnp.minimum(gid, N_GROUP - 1)


def _dispatch_body(x_hbm, dpos_hbm, wpa_hbm, wpb_hbm, xd_hbm, wda_hbm,
                   wdb_hbm, xrows, sem, *bufs):
    idx4 = bufs[0:NSLOT]
    aw4 = bufs[NSLOT:2 * NSLOT]
    bw4 = bufs[2 * NSLOT:3 * NSLOT]
    wid = lax.axis_index("s") * NC + lax.axis_index("c")
    base_t = wid * TPW
    pltpu.sync_copy(x_hbm.at[pl.ds(base_t, TPW)], xrows)
    copies = []
    for s in range(NSLOT):
        pltpu.sync_copy(dpos_hbm.at[s, pl.ds(base_t, TPW)], idx4[s])
        pltpu.sync_copy(wpa_hbm.at[s, pl.ds(base_t, TPW)], aw4[s])
        pltpu.sync_copy(wpb_hbm.at[s, pl.ds(base_t, TPW)], bw4[s])
        copies.append(pltpu.async_copy(xrows, xd_hbm.at[idx4[s]], sem))
        copies.append(pltpu.async_copy(aw4[s], wda_hbm.at[idx4[s]], sem))
        copies.append(pltpu.async_copy(bw4[s], wdb_hbm.at[idx4[s]], sem))
    for c in copies:
        c.wait()


def _gmm_body(gid_ref, xd_ref, w1_ref, w3_ref, w2_ref, wda_ref, wdb_ref,
              y_ref):
    half = pl.program_id(1)
    xb = xd_ref[...].astype(jnp.bfloat16)
    wd = jnp.where(half == 0, wda_ref[...], wdb_ref[...])
    u = jax.lax.dot_general(
        xb, w1_ref[0].astype(jnp.bfloat16), (((1,), (1,)), ((), ())),
        preferred_element_type=jnp.float32)
    v = jax.lax.dot_general(
        xb, w3_ref[0].astype(jnp.bfloat16), (((1,), (1,)), ((), ())),
        preferred_element_type=jnp.float32)
    h = (u * jax.nn.sigmoid(u)) * v * wd
    yh = jax.lax.dot_general(
        h.astype(jnp.bfloat16), w2_ref[0].astype(jnp.bfloat16),
        (((1,), (1,)), ((), ())),
        preferred_element_type=jnp.float32)

    @pl.when(half == 0)
    def _init():
        y_ref[...] = yh

    @pl.when(half != 0)
    def _acc():
        y_ref[...] += yh


def _combine_body(y_hbm, dpos_hbm, out_hbm, idxc, acc, buf, sem):
    wid = lax.axis_index("s") * NC + lax.axis_index("c")
    base_t = wid * TPW
    for c in range(TPW // CH):
        for s in range(NSLOT):
            pltpu.sync_copy(
                dpos_hbm.at[s, pl.ds(base_t + c * CH, CH)], idxc)
            dst = acc if s == 0 else buf
            pltpu.async_copy(y_hbm.at[idxc], dst, sem).wait()
            if s > 0:
                def _add_row(r, carry):
                    for j in range(D // 16):
                        sl = pl.ds(16 * j, 16)
                        acc[r, sl] = acc[r, sl] + buf[r, sl]
                    return carry

                lax.fori_loop(0, CH, _add_row, 0)
        pltpu.sync_copy(acc, out_hbm.at[pl.ds(base_t + c * CH, CH)])


def _shared_body(x_ref, sw1_ref, sw3_ref, sw2_ref, acc_ref, out_ref):
    u = jax.lax.dot_general(
        x_ref[...], sw1_ref[...].astype(jnp.bfloat16), (((1,), (1,)), ((), ())),
        preferred_element_type=jnp.float32)
    v = jax.lax.dot_general(
        x_ref[...], sw3_ref[...].astype(jnp.bfloat16), (((1,), (1,)), ((), ())),
        preferred_element_type=jnp.float32)
    h = (u * jax.nn.sigmoid(u)) * v
    y = jax.lax.dot_general(
        h.astype(jnp.bfloat16), sw2_ref[...].astype(jnp.bfloat16),
        (((1,), (1,)), ((), ())),
        preferred_element_type=jnp.float32)
    out_ref[...] = acc_ref[...] + y


@jax.jit
def kernel(x, gate_w, gate_bias, w1, w3, w2, sw1, sw3, sw2):
    gmask, w, counts = pl.pallas_call(
        _route_a_body,
        grid=(NTB,),
        in_specs=[
            pl.BlockSpec((TB, D), lambda i: (i, 0)),
            pl.BlockSpec((E, D), lambda i: (0, 0)),
            pl.BlockSpec((1, E), lambda i: (0, 0)),
        ],
        out_specs=(
            pl.BlockSpec((TB, N_GROUP), lambda i: (i, 0)),
            pl.BlockSpec((TB, E), lambda i: (i, 0)),
            pl.BlockSpec((1, 1, N_GROUP), lambda i: (i, 0, 0)),
        ),
        out_shape=(
            jax.ShapeDtypeStruct((T, N_GROUP), jnp.float32),
            jax.ShapeDtypeStruct((T, E), jnp.float32),
            jax.ShapeDtypeStruct((NTB, 1, N_GROUP), jnp.float32),
        ),
    )(x, gate_w, gate_bias.reshape(1, E))

    dpos, wpa, wpb, gid = pl.pallas_call(
        _route_b_body,
        grid=(NTB,),
        in_specs=[
            pl.BlockSpec((NTB, 1, N_GROUP), lambda i: (0, 0, 0)),
            pl.BlockSpec((TB, N_GROUP), lambda i: (i, 0)),
            pl.BlockSpec((TB, E), lambda i: (i, 0)),
        ],
        out_specs=(
            pl.BlockSpec((NSLOT, TB), lambda i: (0, i)),
            pl.BlockSpec((NSLOT, TB), lambda i: (0, i)),
            pl.BlockSpec((NSLOT, TB), lambda i: (0, i)),
            pl.BlockSpec((1, 64), lambda i: (0, 0)),
        ),
        out_shape=(
            jax.ShapeDtypeStruct((NSLOT, T), jnp.int32),
            jax.ShapeDtypeStruct((NSLOT, T), jnp.float32),
            jax.ShapeDtypeStruct((NSLOT, T), jnp.float32),
            jax.ShapeDtypeStruct((1, 64), jnp.int32),
        ),
    )(counts, gmask, w)

    mesh = plsc.VectorSubcoreMesh(core_axis_name="c", subcore_axis_name="s")
    dispatch = pl.kernel(
        _dispatch_body,
        mesh=mesh,
        out_type=(
            jax.ShapeDtypeStruct((PPAD, D), jnp.float32),
            jax.ShapeDtypeStruct((PPAD,), jnp.float32),
            jax.ShapeDtypeStruct((PPAD,), jnp.float32),
        ),
        scratch_types=(
            [pltpu.VMEM((TPW, D), jnp.float32),
             pltpu.SemaphoreType.DMA]
            + [pltpu.VMEM((TPW,), jnp.int32) for _ in range(NSLOT)]
            + [pltpu.VMEM((TPW,), jnp.float32) for _ in range(2 * NSLOT)]
        ),
    )
    xd, wda, wdb = dispatch(x, dpos, wpa, wpb)

    y = pl.pallas_call(
        _gmm_body,
        grid_spec=pltpu.PrefetchScalarGridSpec(
            num_scalar_prefetch=1,
            grid=(NBLK, 2),
            in_specs=[
                pl.BlockSpec((BR, D), lambda i, h, gid: (i, 0)),
                pl.BlockSpec(
                    (1, I, D), lambda i, h, gid: (2 * gid[0, i] + h, 0, 0)),
                pl.BlockSpec(
                    (1, I, D), lambda i, h, gid: (2 * gid[0, i] + h, 0, 0)),
                pl.BlockSpec(
                    (1, D, I), lambda i, h, gid: (2 * gid[0, i] + h, 0, 0)),
                pl.BlockSpec((BR, 1), lambda i, h, gid: (i, 0)),
                pl.BlockSpec((BR, 1), lambda i, h, gid: (i, 0)),
            ],
            out_specs=pl.BlockSpec((BR, D), lambda i, h, gid: (i, 0)),
        ),
        out_shape=jax.ShapeDtypeStruct((PPAD, D), jnp.float32),
    )(gid, xd, w1, w3, w2, wda.reshape(PPAD, 1), wdb.reshape(PPAD, 1))

    combine = pl.kernel(
        _combine_body,
        mesh=plsc.VectorSubcoreMesh(core_axis_name="c", subcore_axis_name="s"),
        out_type=jax.ShapeDtypeStruct((T, D), jnp.float32),
        scratch_types=[
            pltpu.VMEM((CH,), jnp.int32),
            pltpu.VMEM((CH, D), jnp.float32),
            pltpu.VMEM((CH, D), jnp.float32),
            pltpu.SemaphoreType.DMA,
        ],
    )
    out_moe = combine(y, dpos)

    out = pl.pallas_call(
        _shared_body,
        input_output_aliases={4: 0},
        out_shape=jax.ShapeDtypeStruct((T, D), jnp.float32),
    )(x.astype(jnp.bfloat16), sw1, sw3, sw2, out_moe)
    return out


# speedup vs baseline: 2.3474x; 2.3474x over previous
import functools

import jax
import jax.numpy as jnp
from jax.experimental import pallas as pl
from jax.experimental.pallas import tpu as pltpu

T = 2048
D = 1024
I = 512
E = 16
K = 8
N_GROUP = 8
TOPK_GROUP = 4
RSF = 2.5
NE = E + 1
WCOLS = 32


def _routing_body(x_ref, gw_ref, gb_ref, wt_ref):
    xb = x_ref[...].astype(jnp.bfloat16)
    gwb = gw_ref[...].astype(jnp.bfloat16)
    logits = jax.lax.dot_general(
        xb, gwb, (((1,), (1,)), ((), ())),
        preferred_element_type=jnp.float32)
    scores = jax.nn.sigmoid(logits)
    sfc = scores + gb_ref[...]

    ecol = jax.lax.broadcasted_iota(jnp.int32, (T, E), 1)
    gs = jnp.concatenate(
        [jnp.sum(jnp.where(ecol // 2 == g, sfc, 0.0), axis=1, keepdims=True)
         for g in range(N_GROUP)], axis=1)

    col = jax.lax.broadcasted_iota(jnp.int32, (T, N_GROUP), 1)
    rank = jnp.zeros((T, N_GROUP), jnp.int32)
    for gp in range(N_GROUP):
        other = gs[:, gp:gp + 1]
        gt = (other > gs).astype(jnp.int32)
        tie = jnp.logical_and(other == gs, gp < col).astype(jnp.int32)
        rank = rank + gt + tie
    gmask = (rank < TOPK_GROUP).astype(jnp.float32)

    e_iota = jax.lax.broadcasted_iota(jnp.int32, (N_GROUP, E), 0)
    g_iota = jax.lax.broadcasted_iota(jnp.int32, (N_GROUP, E), 1)
    pmt = (g_iota // 2 == e_iota).astype(jnp.float32)
    emask = jax.lax.dot_general(
        gmask, pmt, (((1,), (0,)), ((), ())),
        preferred_element_type=jnp.float32)

    w = scores * emask
    denom = jnp.sum(w, axis=1, keepdims=True) + 1e-20
    w = w * (RSF / denom)

    tail_iota = jax.lax.broadcasted_iota(jnp.int32, (T, WCOLS - E), 1)
    tail = (tail_iota == 0).astype(jnp.float32)
    wt_ref[...] = jnp.concatenate([w, tail], axis=1)


def _swiglu(x, w1, w3, w2, scale):
    u = jax.lax.dot_general(
        x, w1.astype(jnp.bfloat16), (((1,), (1,)), ((), ())),
        preferred_element_type=jnp.float32)
    v = jax.lax.dot_general(
        x, w3.astype(jnp.bfloat16), (((1,), (1,)), ((), ())),
        preferred_element_type=jnp.float32)
    h = (u * jax.nn.sigmoid(u)) * v * scale
    return jax.lax.dot_general(
        h.astype(jnp.bfloat16), w2.astype(jnp.bfloat16),
        (((1,), (1,)), ((), ())),
        preferred_element_type=jnp.float32)


def _expert_body(wt_ref, x_ref, w1_ref, w3_ref, w2_ref, out_ref):
    e = pl.program_id(0)
    sel = (jax.lax.broadcasted_iota(jnp.int32, (1, WCOLS), 1) == e)
    we = jnp.sum(wt_ref[...] * sel.astype(jnp.float32), axis=1,
                 keepdims=True)
    y = _swiglu(x_ref[...], w1_ref[0], w3_ref[0], w2_ref[0], we)

    @pl.when(e == 0)
    def _init():
        out_ref[...] = y

    @pl.when(e != 0)
    def _acc():
        out_ref[...] += y


def _shared_body(x_ref, sw1_ref, sw3_ref, sw2_ref, acc_ref, out_ref):
    one = jnp.ones((T, 1), jnp.float32)
    y = _swiglu(x_ref[...], sw1_ref[...], sw3_ref[...], sw2_ref[...], one)
    out_ref[...] = acc_ref[...] + y


@jax.jit
def kernel(x, gate_w, gate_bias, w1, w3, w2, sw1, sw3, sw2):
    wt = pl.pallas_call(
        _routing_body,
        out_shape=jax.ShapeDtypeStruct((T, WCOLS), jnp.float32),
    )(x, gate_w, gate_bias.reshape(1, E))

    xb = x.astype(jnp.bfloat16)

    routed = pl.pallas_call(
        _expert_body,
        grid=(E,),
        in_specs=[
            pl.BlockSpec((T, WCOLS), lambda e: (0, 0)),
            pl.BlockSpec((T, D), lambda e: (0, 0)),
            pl.BlockSpec((1, I, D), lambda e: (e, 0, 0)),
            pl.BlockSpec((1, I, D), lambda e: (e, 0, 0)),
            pl.BlockSpec((1, D, I), lambda e: (e, 0, 0)),
        ],
        out_specs=pl.BlockSpec((T, D), lambda e: (0, 0)),
        out_shape=jax.ShapeDtypeStruct((T, D), jnp.float32),
    )(wt, xb, w1, w3, w2)

    out = pl.pallas_call(
        _shared_body,
        input_output_aliases={4: 0},
        out_shape=jax.ShapeDtypeStruct((T, D), jnp.float32),
    )(xb, sw1, sw3, sw2, routed)
    return out


# speedup vs baseline: 2.6246x; 1.1181x over previous
import functools

import jax
import jax.numpy as jnp
from jax.experimental import pallas as pl
from jax.experimental.pallas import tpu as pltpu

T = 2048
D = 1024
I = 512
E = 16
K = 8
N_GROUP = 8
TOPK_GROUP = 4
RSF = 2.5
NE = E + 1
WCOLS = 32


def _routing_body(x_ref, gw_ref, gb_ref, wt_ref):
    xb = x_ref[...].astype(jnp.bfloat16)
    gwb = gw_ref[...].astype(jnp.bfloat16)
    logits = jax.lax.dot_general(
        xb, gwb, (((1,), (1,)), ((), ())),
        preferred_element_type=jnp.float32)
    scores = jax.nn.sigmoid(logits)
    sfc = scores + gb_ref[...]

    ecol = jax.lax.broadcasted_iota(jnp.int32, (T, E), 1)
    gs = jnp.concatenate(
        [jnp.sum(jnp.where(ecol // 2 == g, sfc, 0.0), axis=1, keepdims=True)
         for g in range(N_GROUP)], axis=1)

    col = jax.lax.broadcasted_iota(jnp.int32, (T, N_GROUP), 1)
    rank = jnp.zeros((T, N_GROUP), jnp.int32)
    for gp in range(N_GROUP):
        other = gs[:, gp:gp + 1]
        gt = (other > gs).astype(jnp.int32)
        tie = jnp.logical_and(other == gs, gp < col).astype(jnp.int32)
        rank = rank + gt + tie
    gmask = (rank < TOPK_GROUP).astype(jnp.float32)

    e_iota = jax.lax.broadcasted_iota(jnp.int32, (N_GROUP, E), 0)
    g_iota = jax.lax.broadcasted_iota(jnp.int32, (N_GROUP, E), 1)
    pmt = (g_iota // 2 == e_iota).astype(jnp.float32)
    emask = jax.lax.dot_general(
        gmask, pmt, (((1,), (0,)), ((), ())),
        preferred_element_type=jnp.float32)

    w = scores * emask
    denom = jnp.sum(w, axis=1, keepdims=True) + 1e-20
    w = w * (RSF / denom)

    tail_iota = jax.lax.broadcasted_iota(jnp.int32, (T, WCOLS - E), 1)
    tail = (tail_iota == 0).astype(jnp.float32)
    wt_ref[...] = jnp.concatenate([w, tail], axis=1)


def _swiglu(x, w1, w3, w2, scale):
    u = jax.lax.dot_general(
        x, w1.astype(jnp.bfloat16), (((1,), (1,)), ((), ())),
        preferred_element_type=jnp.float32)
    v = jax.lax.dot_general(
        x, w3.astype(jnp.bfloat16), (((1,), (1,)), ((), ())),
        preferred_element_type=jnp.float32)
    h = (u * jax.nn.sigmoid(u)) * v * scale
    return jax.lax.dot_general(
        h.astype(jnp.bfloat16), w2.astype(jnp.bfloat16),
        (((1,), (1,)), ((), ())),
        preferred_element_type=jnp.float32)


def _expert_body(wt_ref, x_ref, w1_ref, w3_ref, w2_ref, sw1_ref, sw3_ref,
                 sw2_ref, out_ref):
    e = pl.program_id(0)

    @pl.when(e == 0)
    def _init():
        out_ref[...] = jnp.zeros((T, D), jnp.float32)

    @pl.when(e < E)
    def _routed():
        sel = (jax.lax.broadcasted_iota(jnp.int32, (1, WCOLS), 1) == e)
        we = jnp.sum(wt_ref[...] * sel.astype(jnp.float32), axis=1,
                     keepdims=True)
        out_ref[...] += _swiglu(x_ref[...], w1_ref[0], w3_ref[0], w2_ref[0],
                                we)

    @pl.when(e == E)
    def _shared():
        one = jnp.ones((T, 1), jnp.float32)
        out_ref[...] += _swiglu(x_ref[...], sw1_ref[...], sw3_ref[...],
                                sw2_ref[...], one)


@jax.jit
def kernel(x, gate_w, gate_bias, w1, w3, w2, sw1, sw3, sw2):
    wt = pl.pallas_call(
        _routing_body,
        out_shape=jax.ShapeDtypeStruct((T, WCOLS), jnp.float32),
    )(x, gate_w, gate_bias.reshape(1, E))

    xb = x.astype(jnp.bfloat16)

    out = pl.pallas_call(
        _expert_body,
        grid=(E + 1,),
        in_specs=[
            pl.BlockSpec((T, WCOLS), lambda e: (0, 0)),
            pl.BlockSpec((T, D), lambda e: (0, 0)),
            pl.BlockSpec((1, I, D), lambda e: (jnp.minimum(e, E - 1), 0, 0)),
            pl.BlockSpec((1, I, D), lambda e: (jnp.minimum(e, E - 1), 0, 0)),
            pl.BlockSpec((1, D, I), lambda e: (jnp.minimum(e, E - 1), 0, 0)),
            pl.BlockSpec((I, D), lambda e: (0, 0)),
            pl.BlockSpec((I, D), lambda e: (0, 0)),
            pl.BlockSpec((D, I), lambda e: (0, 0)),
        ],
        out_specs=pl.BlockSpec((T, D), lambda e: (0, 0)),
        out_shape=jax.ShapeDtypeStruct((T, D), jnp.float32),
    )(wt, xb, w1, w3, w2, sw1, sw3, sw2)
    return out
